# baseline (speedup 1.0000x reference)
import functools

import jax
import jax.numpy as jnp
from jax import lax
from jax.experimental import pallas as pl
from jax.experimental.pallas import tpu as pltpu
from jax.experimental.pallas import tpu_sc as plsc

N_EXP = 8
DIM = 2048
BLK_T = 2048
NC = 2
NS = 16
NW = NC * NS
LANES = 16
N_SC = 8192
TOK = 16
GRP = 2


def _mm_kernel(x_ref, w_ref, st_ref):
    st_ref[...] = jax.lax.dot_general(
        w_ref[...], x_ref[...], (((1,), (1,)), ((), ())),
        preferred_element_type=jnp.float32,
    )


def _scores_t(x, W, n_tc):
    _, dim = x.shape
    return pl.pallas_call(
        _mm_kernel,
        grid=(n_tc // BLK_T,),
        in_specs=[
            pl.BlockSpec((BLK_T, dim), lambda i: (i, 0)),
            pl.BlockSpec((N_EXP, dim), lambda i: (0, 0)),
        ],
        out_specs=pl.BlockSpec((N_EXP, BLK_T), lambda i: (0, i)),
        out_shape=jax.ShapeDtypeStruct((N_EXP, n_tc), jnp.float32),
    )(x, W)


def _perm(v, ix):
    dn = lax.GatherDimensionNumbers(
        offset_dims=(), collapsed_slice_dims=(0,), start_index_map=(0,)
    )
    return lax.gather(
        v, ix[:, None], dn, (1,),
        mode=lax.GatherScatterMode.PROMISE_IN_BOUNDS,
    )


def _lane_sum(v, bfly_ix):
    for ix in bfly_ix:
        v = v + _perm(v, ix)
    return v


def _top2_16(vs):
    m1 = vs[0]
    i1 = jnp.zeros((LANES,), jnp.int32)
    m2 = jnp.full((LANES,), -jnp.inf, jnp.float32)
    i2 = jnp.zeros((LANES,), jnp.int32)
    for e in range(1, N_EXP):
        v = vs[e]
        ev = jnp.full((LANES,), e, jnp.int32)
        gt1 = v > m1
        gt2 = v > m2
        m2n = jnp.where(gt1, m1, jnp.where(gt2, v, m2))
        i2n = jnp.where(gt1, i1, jnp.where(gt2, ev, i2))
        m1 = jnp.where(gt1, v, m1)
        i1 = jnp.where(gt1, ev, i1)
        m2, i2 = m2n, i2n
    denom = jnp.zeros((LANES,), jnp.float32)
    for e in range(N_EXP):
        denom = denom + jnp.exp(vs[e] - m1)
    w1 = 1.0 / denom
    w2 = jnp.exp(m2 - m1) * w1
    return w1, w2, i1, i2


def _route_scores(s_v, w_v, i_v, chunk):
    def body(t, _):
        off = t * LANES
        vs = [s_v[e, pl.ds(off, LANES)] for e in range(N_EXP)]
        w1, w2, i1, i2 = _top2_16(vs)
        sl = pl.ds(off, LANES)
        w_v[0, sl] = w1
        w_v[1, sl] = w2
        i_v[0, sl] = i1
        i_v[1, sl] = i2
        return 0

    lax.fori_loop(0, chunk // LANES, body, 0)


def _make_route(n_tc):
    chunk = n_tc // NW

    @functools.partial(
        pl.kernel,
        mesh=plsc.VectorSubcoreMesh(core_axis_name="c", subcore_axis_name="s"),
        out_type=[
            jax.ShapeDtypeStruct((n_tc,), jnp.float32),
            jax.ShapeDtypeStruct((n_tc,), jnp.float32),
            jax.ShapeDtypeStruct((n_tc,), jnp.int32),
            jax.ShapeDtypeStruct((n_tc,), jnp.int32),
        ],
        scratch_types=[
            pltpu.VMEM((N_EXP, chunk), jnp.float32),
            pltpu.VMEM((2, chunk), jnp.float32),
            pltpu.VMEM((2, chunk), jnp.int32),
            pltpu.SemaphoreType.DMA,
            pltpu.SemaphoreType.DMA,
        ],
    )
    def route(st_hbm, w1_hbm, w2_hbm, i1_hbm, i2_hbm, s_v, w_v, i_v,
              in_sem, out_sem):
        wid = lax.axis_index("s") * NC + lax.axis_index("c")
        base = wid * chunk
        copies = [
            pltpu.async_copy(
                st_hbm.at[pl.ds(e * n_tc + base, chunk)], s_v.at[e], in_sem
            )
            for e in range(N_EXP)
        ]
        for c in copies:
            c.wait()
        _route_scores(s_v, w_v, i_v, chunk)
        rows = pl.ds(base, chunk)
        outs = [
            pltpu.async_copy(w_v.at[0], w1_hbm.at[rows], out_sem),
            pltpu.async_copy(w_v.at[1], w2_hbm.at[rows], out_sem),
            pltpu.async_copy(i_v.at[0], i1_hbm.at[rows], out_sem),
            pltpu.async_copy(i_v.at[1], i2_hbm.at[rows], out_sem),
        ]
        for c in outs:
            c.wait()

    return route


def _make_scgate(n_tokens, n_tc):
    chunk = N_SC // NW
    tiles = chunk // TOK
    pairs = tiles // 2

    @functools.partial(
        pl.kernel,
        mesh=plsc.VectorSubcoreMesh(core_axis_name="c", subcore_axis_name="s"),
        out_type=[
            jax.ShapeDtypeStruct((N_SC,), jnp.float32),
            jax.ShapeDtypeStruct((N_SC,), jnp.float32),
            jax.ShapeDtypeStruct((N_SC,), jnp.int32),
            jax.ShapeDtypeStruct((N_SC,), jnp.int32),
        ],
        scratch_types=[
            pltpu.VMEM((2, TOK, DIM), jnp.float32),
            pltpu.VMEM((N_EXP, DIM), jnp.float32),
            pltpu.VMEM((2, chunk), jnp.float32),
            pltpu.VMEM((2, chunk), jnp.int32),
            pltpu.SemaphoreType.DMA,
            pltpu.SemaphoreType.DMA,
            pltpu.SemaphoreType.DMA,
        ],
    )
    def scgate(x_hbm, w_hbm, w1_hbm, w2_hbm, i1_hbm, i2_hbm,
               x_v, w_v, wo_v, io_v, sem_a, sem_b, out_sem):
        wid = lax.axis_index("s") * NC + lax.axis_index("c")
        base_row = n_tc + wid * chunk
        pltpu.sync_copy(w_hbm, w_v)
        lane = lax.iota(jnp.int32, LANES)
        bfly_ix = [lane ^ k for k in (1, 2, 4, 8)]

        def tile_rows(t):
            return x_hbm.at[pl.ds(base_row + t * TOK, TOK)]

        def process(buf, tok0):
            vouts = [jnp.zeros((LANES,), jnp.float32) for _ in range(N_EXP)]
            for g in range(TOK // GRP):

                def dbody(d, accs):
                    off = d * LANES
                    def rnd(v):
                        u = lax.bitcast_convert_type(v, jnp.uint32)
                        u = u + 0x7FFF + ((u >> 16) & 1)
                        u = u & jnp.uint32(0xFFFF0000)
                        return lax.bitcast_convert_type(u, jnp.float32)

                    xs = [
                        rnd(x_v[buf, g * GRP + j, pl.ds(off, LANES)])
                        for j in range(GRP)
                    ]
                    ws = [
                        rnd(w_v[e, pl.ds(off, LANES)])
                        for e in range(N_EXP)
                    ]
                    return tuple(
                        accs[j * N_EXP + e] + xs[j] * ws[e]
                        for j in range(GRP)
                        for e in range(N_EXP)
                    )

                zeros = (jnp.zeros((LANES,), jnp.float32),) * (GRP * N_EXP)
                accs = lax.fori_loop(0, DIM // LANES, dbody, zeros)
                for j in range(GRP):
                    lane_mask = lane == (g * GRP + j)
                    for e in range(N_EXP):
                        s = _lane_sum(accs[j * N_EXP + e], bfly_ix)
                        vouts[e] = jnp.where(lane_mask, s, vouts[e])
            w1, w2, i1, i2 = _top2_16(vouts)
            sl = pl.ds(tok0, LANES)
            wo_v[0, sl] = w1
            wo_v[1, sl] = w2
            io_v[0, sl] = i1
            io_v[1, sl] = i2

        first = pltpu.async_copy(tile_rows(0), x_v.at[0], sem_a)

        def pair_body(p, _):
            t0 = p * 2
            nxt = pltpu.async_copy(tile_rows(t0 + 1), x_v.at[1], sem_b)
            pltpu.make_async_copy(tile_rows(t0), x_v.at[0], sem_a).wait()
            process(0, t0 * TOK)
            wrap = lax.rem(t0 + 2, tiles)
            pltpu.async_copy(tile_rows(wrap), x_v.at[0], sem_a)
            nxt.wait()
            process(1, (t0 + 1) * TOK)
            return 0

        lax.fori_loop(0, pairs, pair_body, 0)
        pltpu.make_async_copy(tile_rows(0), x_v.at[0], sem_a).wait()

        rows = pl.ds(wid * chunk, chunk)
        outs = [
            pltpu.async_copy(wo_v.at[0], w1_hbm.at[rows], out_sem),
            pltpu.async_copy(wo_v.at[1], w2_hbm.at[rows], out_sem),
            pltpu.async_copy(io_v.at[0], i1_hbm.at[rows], out_sem),
            pltpu.async_copy(io_v.at[1], i2_hbm.at[rows], out_sem),
        ]
        for c in outs:
            c.wait()

    return scgate


@jax.jit
def kernel(x, W):
    n_tokens, _ = x.shape
    n_tc = n_tokens - N_SC
    sg = _make_scgate(n_tokens, n_tc)(x, W)
    st = _scores_t(x, W, n_tc)
    rt = _make_route(n_tc)(st.reshape(-1))
    w1 = jnp.concatenate([rt[0], sg[0]])
    w2 = jnp.concatenate([rt[1], sg[1]])
    i1 = jnp.concatenate([rt[2], sg[2]])
    i2 = jnp.concatenate([rt[3], sg[3]])
    return jnp.stack([w1, w2], axis=1), jnp.stack([i1, i2], axis=1)


# speedup vs baseline: 1.7202x; 1.7202x over previous
import functools

import jax
import jax.numpy as jnp
from jax import lax
from jax.experimental import pallas as pl
from jax.experimental.pallas import tpu as pltpu
from jax.experimental.pallas import tpu_sc as plsc

N_EXP = 8
DIM = 2048
BLK_T = 2048
NC = 2
NS = 16
NW = NC * NS
LANES = 16
N_SC = 4096
TOK = 16
GRP = 4


def _mm_kernel(x_ref, w_ref, st_ref):
    st_ref[...] = jax.lax.dot_general(
        w_ref[...], x_ref[...], (((1,), (1,)), ((), ())),
        preferred_element_type=jnp.float32,
    )


def _scores_t(x, W, n_tc):
    _, dim = x.shape
    return pl.pallas_call(
        _mm_kernel,
        grid=(n_tc // BLK_T,),
        in_specs=[
            pl.BlockSpec((BLK_T, dim), lambda i: (i, 0)),
            pl.BlockSpec((N_EXP, dim), lambda i: (0, 0)),
        ],
        out_specs=pl.BlockSpec((N_EXP, BLK_T), lambda i: (0, i)),
        out_shape=jax.ShapeDtypeStruct((N_EXP, n_tc), jnp.float32),
    )(x, W)


def _perm(v, ix):
    dn = lax.GatherDimensionNumbers(
        offset_dims=(), collapsed_slice_dims=(0,), start_index_map=(0,)
    )
    return lax.gather(
        v, ix[:, None], dn, (1,),
        mode=lax.GatherScatterMode.PROMISE_IN_BOUNDS,
    )


def _lane_sum(v, bfly_ix):
    for ix in bfly_ix:
        v = v + _perm(v, ix)
    return v


def _top2_16(vs):
    m1 = vs[0]
    i1 = jnp.zeros((LANES,), jnp.int32)
    m2 = jnp.full((LANES,), -jnp.inf, jnp.float32)
    i2 = jnp.zeros((LANES,), jnp.int32)
    for e in range(1, N_EXP):
        v = vs[e]
        ev = jnp.full((LANES,), e, jnp.int32)
        gt1 = v > m1
        gt2 = v > m2
        m2n = jnp.where(gt1, m1, jnp.where(gt2, v, m2))
        i2n = jnp.where(gt1, i1, jnp.where(gt2, ev, i2))
        m1 = jnp.where(gt1, v, m1)
        i1 = jnp.where(gt1, ev, i1)
        m2, i2 = m2n, i2n
    denom = jnp.zeros((LANES,), jnp.float32)
    for e in range(N_EXP):
        denom = denom + jnp.exp(vs[e] - m1)
    w1 = 1.0 / denom
    w2 = jnp.exp(m2 - m1) * w1
    return w1, w2, i1, i2


def _route_scores(s_v, w_v, i_v, chunk):
    def body(t, _):
        off = t * LANES
        vs = [s_v[e, pl.ds(off, LANES)] for e in range(N_EXP)]
        w1, w2, i1, i2 = _top2_16(vs)
        sl = pl.ds(off, LANES)
        w_v[0, sl] = w1
        w_v[1, sl] = w2
        i_v[0, sl] = i1
        i_v[1, sl] = i2
        return 0

    lax.fori_loop(0, chunk // LANES, body, 0)


def _make_route(n_tc):
    chunk = n_tc // NW

    @functools.partial(
        pl.kernel,
        mesh=plsc.VectorSubcoreMesh(core_axis_name="c", subcore_axis_name="s"),
        out_type=[
            jax.ShapeDtypeStruct((n_tc,), jnp.float32),
            jax.ShapeDtypeStruct((n_tc,), jnp.float32),
            jax.ShapeDtypeStruct((n_tc,), jnp.int32),
            jax.ShapeDtypeStruct((n_tc,), jnp.int32),
        ],
        scratch_types=[
            pltpu.VMEM((N_EXP, chunk), jnp.float32),
            pltpu.VMEM((2, chunk), jnp.float32),
            pltpu.VMEM((2, chunk), jnp.int32),
            pltpu.SemaphoreType.DMA,
            pltpu.SemaphoreType.DMA,
        ],
    )
    def route(st_hbm, w1_hbm, w2_hbm, i1_hbm, i2_hbm, s_v, w_v, i_v,
              in_sem, out_sem):
        wid = lax.axis_index("s") * NC + lax.axis_index("c")
        base = wid * chunk
        copies = [
            pltpu.async_copy(
                st_hbm.at[pl.ds(e * n_tc + base, chunk)], s_v.at[e], in_sem
            )
            for e in range(N_EXP)
        ]
        for c in copies:
            c.wait()
        _route_scores(s_v, w_v, i_v, chunk)
        rows = pl.ds(base, chunk)
        outs = [
            pltpu.async_copy(w_v.at[0], w1_hbm.at[rows], out_sem),
            pltpu.async_copy(w_v.at[1], w2_hbm.at[rows], out_sem),
            pltpu.async_copy(i_v.at[0], i1_hbm.at[rows], out_sem),
            pltpu.async_copy(i_v.at[1], i2_hbm.at[rows], out_sem),
        ]
        for c in outs:
            c.wait()

    return route


def _make_scgate(n_tokens, n_tc):
    chunk = N_SC // NW
    tiles = chunk // TOK
    pairs = tiles // 2

    @functools.partial(
        pl.kernel,
        mesh=plsc.VectorSubcoreMesh(core_axis_name="c", subcore_axis_name="s"),
        out_type=[
            jax.ShapeDtypeStruct((N_SC,), jnp.float32),
            jax.ShapeDtypeStruct((N_SC,), jnp.float32),
            jax.ShapeDtypeStruct((N_SC,), jnp.int32),
            jax.ShapeDtypeStruct((N_SC,), jnp.int32),
        ],
        scratch_types=[
            pltpu.VMEM((2, TOK, DIM), jnp.float32),
            pltpu.VMEM((N_EXP, DIM), jnp.float32),
            pltpu.VMEM((2, chunk), jnp.float32),
            pltpu.VMEM((2, chunk), jnp.int32),
            pltpu.SemaphoreType.DMA,
            pltpu.SemaphoreType.DMA,
            pltpu.SemaphoreType.DMA,
        ],
    )
    def scgate(x_hbm, w_hbm, w1_hbm, w2_hbm, i1_hbm, i2_hbm,
               x_v, w_v, wo_v, io_v, sem_a, sem_b, out_sem):
        wid = lax.axis_index("s") * NC + lax.axis_index("c")
        base_row = n_tc + wid * chunk
        pltpu.sync_copy(w_hbm, w_v)
        lane = lax.iota(jnp.int32, LANES)
        bfly_ix = [lane ^ k for k in (1, 2, 4, 8)]

        def rnd(v):
            u = lax.bitcast_convert_type(v, jnp.uint32)
            u = u + 0x7FFF + ((u >> 16) & 1)
            u = u & jnp.uint32(0xFFFF0000)
            return lax.bitcast_convert_type(u, jnp.float32)

        def wrnd_body(d, _):
            sl = pl.ds(d * LANES, LANES)
            for e in range(N_EXP):
                w_v[e, sl] = rnd(w_v[e, sl])
            return 0

        lax.fori_loop(0, DIM // LANES, wrnd_body, 0)

        def tile_rows(t):
            return x_hbm.at[pl.ds(base_row + t * TOK, TOK)]

        def process(buf, tok0):
            vouts = [jnp.zeros((LANES,), jnp.float32) for _ in range(N_EXP)]
            for g in range(TOK // GRP):

                def dbody(d, accs):
                    for u in range(2):
                        off = (d * 2 + u) * LANES
                        xs = [
                            rnd(x_v[buf, g * GRP + j, pl.ds(off, LANES)])
                            for j in range(GRP)
                        ]
                        ws = [w_v[e, pl.ds(off, LANES)] for e in range(N_EXP)]
                        accs = tuple(
                            accs[j * N_EXP + e] + xs[j] * ws[e]
                            for j in range(GRP)
                            for e in range(N_EXP)
                        )
                    return accs

                zeros = (jnp.zeros((LANES,), jnp.float32),) * (GRP * N_EXP)
                accs = lax.fori_loop(0, DIM // (2 * LANES), dbody, zeros)
                for j in range(GRP):
                    lane_mask = lane == (g * GRP + j)
                    for e in range(N_EXP):
                        s = _lane_sum(accs[j * N_EXP + e], bfly_ix)
                        vouts[e] = jnp.where(lane_mask, s, vouts[e])
            w1, w2, i1, i2 = _top2_16(vouts)
            sl = pl.ds(tok0, LANES)
            wo_v[0, sl] = w1
            wo_v[1, sl] = w2
            io_v[0, sl] = i1
            io_v[1, sl] = i2

        first = pltpu.async_copy(tile_rows(0), x_v.at[0], sem_a)

        def pair_body(p, _):
            t0 = p * 2
            nxt = pltpu.async_copy(tile_rows(t0 + 1), x_v.at[1], sem_b)
            pltpu.make_async_copy(tile_rows(t0), x_v.at[0], sem_a).wait()
            process(0, t0 * TOK)
            wrap = lax.rem(t0 + 2, tiles)
            pltpu.async_copy(tile_rows(wrap), x_v.at[0], sem_a)
            nxt.wait()
            process(1, (t0 + 1) * TOK)
            return 0

        lax.fori_loop(0, pairs, pair_body, 0)
        pltpu.make_async_copy(tile_rows(0), x_v.at[0], sem_a).wait()

        rows = pl.ds(wid * chunk, chunk)
        outs = [
            pltpu.async_copy(wo_v.at[0], w1_hbm.at[rows], out_sem),
            pltpu.async_copy(wo_v.at[1], w2_hbm.at[rows], out_sem),
            pltpu.async_copy(io_v.at[0], i1_hbm.at[rows], out_sem),
            pltpu.async_copy(io_v.at[1], i2_hbm.at[rows], out_sem),
        ]
        for c in outs:
            c.wait()

    return scgate


@jax.jit
def kernel(x, W):
    n_tokens, _ = x.shape
    n_tc = n_tokens - N_SC
    sg = _make_scgate(n_tokens, n_tc)(x, W)
    st = _scores_t(x, W, n_tc)
    rt = _make_route(n_tc)(st.reshape(-1))
    w1 = jnp.concatenate([rt[0], sg[0]])
    w2 = jnp.concatenate([rt[1], sg[1]])
    i1 = jnp.concatenate([rt[2], sg[2]])
    i2 = jnp.concatenate([rt[3], sg[3]])
    return jnp.stack([w1, w2], axis=1), jnp.stack([i1, i2], axis=1)


# speedup vs baseline: 3.5043x; 2.0371x over previous
import functools

import jax
import jax.numpy as jnp
from jax import lax
from jax.experimental import pallas as pl
from jax.experimental.pallas import tpu as pltpu
from jax.experimental.pallas import tpu_sc as plsc

N_EXP = 8
BLK_T = 2048
NC = 2
NS = 16
NW = NC * NS
LANES = 16


def _mm_kernel(x_ref, w_ref, st_ref):
    st_ref[...] = jax.lax.dot_general(
        w_ref[...], x_ref[...], (((1,), (1,)), ((), ())),
        preferred_element_type=jnp.float32,
    )


def _scores_t(x, W):
    n_tokens, dim = x.shape
    return pl.pallas_call(
        _mm_kernel,
        grid=(n_tokens // BLK_T,),
        in_specs=[
            pl.BlockSpec((BLK_T, dim), lambda i: (i, 0)),
            pl.BlockSpec((N_EXP, dim), lambda i: (0, 0)),
        ],
        out_specs=pl.BlockSpec((N_EXP, BLK_T), lambda i: (0, i)),
        out_shape=jax.ShapeDtypeStruct((N_EXP, n_tokens), jnp.float32),
    )(x, W)


def _make_route(n_tokens):
    chunk = n_tokens // NW

    @functools.partial(
        pl.kernel,
        mesh=plsc.VectorSubcoreMesh(core_axis_name="c", subcore_axis_name="s"),
        out_type=[
            jax.ShapeDtypeStruct((n_tokens,), jnp.float32),
            jax.ShapeDtypeStruct((n_tokens,), jnp.float32),
            jax.ShapeDtypeStruct((n_tokens,), jnp.int32),
            jax.ShapeDtypeStruct((n_tokens,), jnp.int32),
        ],
        scratch_types=[
            pltpu.VMEM((N_EXP, chunk), jnp.float32),
            pltpu.VMEM((2, chunk), jnp.float32),
            pltpu.VMEM((2, chunk), jnp.int32),
            pltpu.SemaphoreType.DMA,
            pltpu.SemaphoreType.DMA,
        ],
    )
    def route(st_hbm, w1_hbm, w2_hbm, i1_hbm, i2_hbm, s_v, w_v, i_v,
              in_sem, out_sem):
        wid = lax.axis_index("s") * NC + lax.axis_index("c")
        base = wid * chunk
        copies = [
            pltpu.async_copy(
                st_hbm.at[pl.ds(e * n_tokens + base, chunk)], s_v.at[e], in_sem
            )
            for e in range(N_EXP)
        ]
        for c in copies:
            c.wait()

        def body(t, _):
            off = t * LANES
            vs = [s_v[e, pl.ds(off, LANES)] for e in range(N_EXP)]
            m1 = vs[0]
            i1 = jnp.zeros((LANES,), jnp.int32)
            m2 = jnp.full((LANES,), -jnp.inf, jnp.float32)
            i2 = jnp.zeros((LANES,), jnp.int32)
            for e in range(1, N_EXP):
                v = vs[e]
                ev = jnp.full((LANES,), e, jnp.int32)
                gt1 = v > m1
                gt2 = v > m2
                m2n = jnp.where(gt1, m1, jnp.where(gt2, v, m2))
                i2n = jnp.where(gt1, i1, jnp.where(gt2, ev, i2))
                m1 = jnp.where(gt1, v, m1)
                i1 = jnp.where(gt1, ev, i1)
                m2, i2 = m2n, i2n
            denom = jnp.zeros((LANES,), jnp.float32)
            for e in range(N_EXP):
                denom = denom + jnp.exp(vs[e] - m1)
            w1 = 1.0 / denom
            w2 = jnp.exp(m2 - m1) * w1
            sl = pl.ds(off, LANES)
            w_v[0, sl] = w1
            w_v[1, sl] = w2
            i_v[0, sl] = i1
            i_v[1, sl] = i2
            return 0

        lax.fori_loop(0, chunk // LANES, body, 0)
        rows = pl.ds(base, chunk)
        outs = [
            pltpu.async_copy(w_v.at[0], w1_hbm.at[rows], out_sem),
            pltpu.async_copy(w_v.at[1], w2_hbm.at[rows], out_sem),
            pltpu.async_copy(i_v.at[0], i1_hbm.at[rows], out_sem),
            pltpu.async_copy(i_v.at[1], i2_hbm.at[rows], out_sem),
        ]
        for c in outs:
            c.wait()

    return route


@jax.jit
def kernel(x, W):
    n_tokens, _ = x.shape
    st = _scores_t(x, W)
    w1, w2, i1, i2 = _make_route(n_tokens)(st.reshape(-1))
    return jnp.stack([w1, w2], axis=1), jnp.stack([i1, i2], axis=1)
